# baseline (speedup 1.0000x reference)
import functools

import jax
import jax.numpy as jnp
from jax import lax
from jax.experimental import pallas as pl
from jax.experimental.pallas import tpu as pltpu
from jax.experimental.pallas import tpu_sc as plsc

E = 100000
R = 500
S_DIM = 96
T_DIM = 32
B = 4096

NC = 2
NS = 16
NW = NC * NS
BPW = B // NW

TC_BLK = 512


def _sc_gather_body(*refs):
    (ent_embs, ent_transfer,
     yf, mf, df, yp, mp, dp, ya, ma, da,
     rel_embs, rel_transfer,
     heads, tails, rels) = refs[:16]
    outs = refs[16:40]
    (idx_h, idx_t, idx_r,
     b96_0, b96_1, b32_0, b32_1, b128_0, b128_1,
     sem0, sem1) = refs[40:]

    cid = lax.axis_index("c")
    sid = lax.axis_index("s")
    wid = sid * NC + cid
    base = wid * BPW

    pltpu.sync_copy(heads.at[pl.ds(base, BPW)], idx_h)
    pltpu.sync_copy(tails.at[pl.ds(base, BPW)], idx_t)
    pltpu.sync_copy(rels.at[pl.ds(base, BPW)], idx_r)

    time_tabs = (yf, mf, df, yp, mp, dp, ya, ma, da)

    g96 = [(ent_embs, idx_h, outs[0]), (ent_transfer, idx_h, outs[1]),
           (ent_embs, idx_t, outs[2]), (ent_transfer, idx_t, outs[3])]
    g32 = [(time_tabs[k], idx_h, outs[4 + k]) for k in range(9)] + \
          [(time_tabs[k], idx_t, outs[13 + k]) for k in range(9)]
    g128 = [(rel_embs, idx_r, outs[22]), (rel_transfer, idx_r, outs[23])]

    sems = (sem0, sem1)

    def run_group(items, bufs):
        copies = [None, None]
        c = pltpu.make_async_copy(items[0][0].at[items[0][1]], bufs[0], sems[0])
        c.start()
        copies[0] = c
        for i, (_, _, out) in enumerate(items):
            if i + 1 < len(items):
                tab2, idx2, _ = items[i + 1]
                c2 = pltpu.make_async_copy(tab2.at[idx2], bufs[(i + 1) % 2],
                                           sems[(i + 1) % 2])
                c2.start()
                copies[(i + 1) % 2] = c2
            copies[i % 2].wait()
            pltpu.sync_copy(bufs[i % 2], out.at[pl.ds(base, BPW)])

    run_group(g96, (b96_0, b96_1))
    run_group(g32, (b32_0, b32_1))
    run_group(g128, (b128_0, b128_1))


@jax.jit
def _sc_gather(ent_embs, ent_transfer, yf, mf, df, yp, mp, dp, ya, ma, da,
               rel_embs, rel_transfer, heads, tails, rels):
    widths = [96, 96, 96, 96] + [32] * 18 + [128, 128]
    out_types = [jax.ShapeDtypeStruct((B, w), jnp.float32) for w in widths]
    mesh = plsc.VectorSubcoreMesh(core_axis_name="c", subcore_axis_name="s")
    scratch = [
        pltpu.VMEM((BPW,), jnp.int32),
        pltpu.VMEM((BPW,), jnp.int32),
        pltpu.VMEM((BPW,), jnp.int32),
        pltpu.VMEM((BPW, 96), jnp.float32),
        pltpu.VMEM((BPW, 96), jnp.float32),
        pltpu.VMEM((BPW, 32), jnp.float32),
        pltpu.VMEM((BPW, 32), jnp.float32),
        pltpu.VMEM((BPW, 128), jnp.float32),
        pltpu.VMEM((BPW, 128), jnp.float32),
        pltpu.SemaphoreType.DMA,
        pltpu.SemaphoreType.DMA,
    ]
    kern = pl.kernel(_sc_gather_body, out_type=out_types, mesh=mesh,
                     scratch_types=scratch,
                     compiler_params=pltpu.CompilerParams(
                         use_tc_tiling_on_sc=False))
    return kern(ent_embs, ent_transfer, yf, mf, df, yp, mp, dp, ya, ma, da,
                rel_embs, rel_transfer, heads, tails, rels)


def _qmul(a, b):
    sa, xa, ya, za = a
    sb, xb, yb, zb = b
    return (sa * sb - xa * xb - ya * yb - za * zb,
            sa * xb + sb * xa + ya * zb - yb * za,
            sa * yb + sb * ya + za * xb - zb * xa,
            sa * zb + sb * za + xa * yb - xb * ya)


def _qnorm(b):
    sb, xb, yb, zb = b
    inv = lax.rsqrt(sb * sb + xb * xb + yb * yb + zb * zb)
    return (sb * inv, xb * inv, yb * inv, zb * inv)


def _split4(v):
    return (v[:, 0:32], v[:, 32:64], v[:, 64:96], v[:, 96:128])


def _tc_body(y_r, m_r, d_r,
             eh, etrh, yfh, mfh, dfh, yph, mph, dph, yah, mah, dah,
             et, etrt, yft, mft, dft, ypt, mpt, dpt, yat, mat, dat,
             r, rtr, out):
    y = y_r[...]
    m = m_r[...]
    d = d_r[...]

    def time_emb(yfr, mfr, dfr, ypr, mpr, dpr, yar, mar, dar):
        return (yar[...] * jnp.sin(yfr[...] * y + ypr[...])
                + mar[...] * jnp.sin(mfr[...] * m + mpr[...])
                + dar[...] * jnp.sin(dfr[...] * d + dpr[...]))

    th = time_emb(yfh, mfh, dfh, yph, mph, dph, yah, mah, dah)
    tt = time_emb(yft, mft, dft, ypt, mpt, dpt, yat, mat, dat)

    ehv = eh[...]
    etrhv = etrh[...]
    etv = et[...]
    etrtv = etrt[...]
    h = (ehv[:, 0:32], ehv[:, 32:64], ehv[:, 64:96], th)
    h_tr = (etrhv[:, 0:32], etrhv[:, 32:64], etrhv[:, 64:96], th)
    t = (etv[:, 0:32], etv[:, 32:64], etv[:, 64:96], tt)
    t_tr = (etrtv[:, 0:32], etrtv[:, 32:64], etrtv[:, 64:96], tt)

    rq = _split4(r[...])
    rtrq = _split4(rtr[...])
    nrtr = _qnorm(rtrq)
    nr = _qnorm(rq)

    h1 = _qmul(_qmul(h, _qnorm(h_tr)), nrtr)
    t1 = _qmul(_qmul(t, _qnorm(t_tr)), nrtr)
    hr = _qmul(h1, nr)

    acc = (hr[0] * t1[0] + hr[1] * t1[1] + hr[2] * t1[2] + hr[3] * t1[3])
    out[...] = jnp.sum(acc, axis=1, keepdims=True)


@jax.jit
def _tc_compute(years, months, days, *gathered):
    grid = (B // TC_BLK,)

    def spec(w):
        return pl.BlockSpec((TC_BLK, w), lambda i: (i, 0))

    widths = [1, 1, 1] + [96, 96] + [32] * 9 + [96, 96] + [32] * 9 + [128, 128]
    in_specs = [spec(w) for w in widths]
    return pl.pallas_call(
        _tc_body,
        grid=grid,
        in_specs=in_specs,
        out_specs=pl.BlockSpec((TC_BLK, 1), lambda i: (i, 0)),
        out_shape=jax.ShapeDtypeStruct((B, 1), jnp.float32),
    )(years.reshape(B, 1), months.reshape(B, 1), days.reshape(B, 1), *gathered)


def kernel(heads, rels, tails, years, months, days, ent_embs, rel_embs,
           ent_transfer, rel_transfer, y_freq, m_freq, d_freq, y_phi, m_phi,
           d_phi, y_amp, m_amp, d_amp):
    heads = heads.astype(jnp.int32)
    tails = tails.astype(jnp.int32)
    rels = rels.astype(jnp.int32)

    g = _sc_gather(ent_embs, ent_transfer, y_freq, m_freq, d_freq, y_phi,
                   m_phi, d_phi, y_amp, m_amp, d_amp, rel_embs, rel_transfer,
                   heads, tails, rels)
    ordered = (g[0], g[1]) + tuple(g[4:13]) + (g[2], g[3]) + tuple(g[13:22]) \
        + (g[22], g[23])
    score = _tc_compute(years, months, days, *ordered)
    return score.reshape(B)
